# baseline (speedup 1.0000x reference)
import functools

import jax
import jax.numpy as jnp
from jax import lax
from jax.experimental import pallas as pl
from jax.experimental.pallas import tpu as pltpu
from jax.experimental.pallas import tpu_sc as plsc

_PLANE = 512 * 512
_NBATCH = 16
_NPIX = _NBATCH * _PLANE
_NC, _NS, _L = 2, 16, 16
_NW = _NC * _NS
_PPW = _NPIX // _NW
_CH = 4096
_NCH = _PPW // _CH
_NV = _CH // _L
_NBINS = 16
_ROWS = _NBINS + 1
_PROW = 192


def _mesh():
    return plsc.VectorSubcoreMesh(
        core_axis_name="c", subcore_axis_name="s",
        num_cores=_NC, num_subcores=_NS)


@functools.partial(
    pl.kernel,
    out_type=jax.ShapeDtypeStruct((_NW * _PROW,), jnp.float32),
    mesh=_mesh(),
    scratch_types=[
        pltpu.VMEM((2 * 9 * _CH,), jnp.float32),
        pltpu.VMEM((_ROWS * _L,), jnp.float32),
        pltpu.VMEM((_ROWS * _L,), jnp.float32),
        pltpu.VMEM((_ROWS * _L,), jnp.float32),
        pltpu.VMEM((_PROW,), jnp.float32),
        pltpu.SemaphoreType.DMA,
        pltpu.SemaphoreType.DMA,
    ],
    compiler_params=pltpu.CompilerParams(needs_layout_passes=False),
)
def _sc_hist(inp_h, pred_h, targ_h, out_h, buf, hc, hp, ht, ob, sem0, sem1):
    w = lax.axis_index("s") * _NC + lax.axis_index("c")
    b = w // 2
    h = w % 2
    base = b * (3 * _PLANE) + h * _PPW
    sems = (sem0, sem1)
    arrs = (inp_h, pred_h, targ_h)

    zero = jnp.zeros((_L,), jnp.float32)
    for i in range(_ROWS):
        hc[pl.ds(i * _L, _L)] = zero
        hp[pl.ds(i * _L, _L)] = zero
        ht[pl.ds(i * _L, _L)] = zero
    for q in range(_PROW // _L):
        ob[pl.ds(q * _L, _L)] = zero

    def copies(g, ph):
        out = []
        for a in range(3):
            for c in range(3):
                src = arrs[a].at[pl.ds(base + c * _PLANE + g * _CH, _CH)]
                dst = buf.at[pl.ds((ph * 9 + a * 3 + c) * _CH, _CH)]
                out.append(pltpu.make_async_copy(src, dst, sems[ph]))
        return out

    def start(g, ph):
        for cp in copies(g, ph):
            cp.start()

    def drain(g, ph):
        for cp in copies(g, ph):
            cp.wait()

    lane = lax.iota(jnp.int32, _L)
    ones = jnp.ones((_L,), jnp.float32)

    def process(ph):
        def body(i, carry):
            def ld(j):
                return buf[pl.ds((ph * 9 + j) * _CH + i * _L, _L)]
            il = 0.299 * ld(0) + 0.587 * ld(1) + 0.114 * ld(2)
            addr = jnp.minimum((il * 16.0).astype(jnp.int32), _ROWS - 1) * _L
            addr = addr + lane
            plu = 0.299 * ld(3) + 0.587 * ld(4) + 0.114 * ld(5)
            tlu = 0.299 * ld(6) + 0.587 * ld(7) + 0.114 * ld(8)
            plsc.addupdate_scatter(hc, [addr], ones)
            plsc.addupdate_scatter(hp, [addr], plu)
            plsc.addupdate_scatter(ht, [addr], tlu)
            return carry
        lax.fori_loop(0, _NV, body, 0)

    start(0, 0)
    start(1, 1)

    def outer(g0, carry):
        for ph in range(2):
            g = g0 * 2 + ph
            drain(g, ph)
            process(ph)

            @pl.when(g + 2 < _NCH)
            def _():
                start(g + 2, ph)
        return carry

    lax.fori_loop(0, _NCH // 2, outer, 0)

    def lane_sums(href):
        acc = jnp.zeros((_L,), jnp.float32)
        for n in range(_NBINS):
            acc = jnp.where(lane == n, jnp.sum(href[pl.ds(n * _L, _L)]), acc)
        return acc

    ob[pl.ds(0, _L)] = lane_sums(hc)
    ob[pl.ds(64, _L)] = lane_sums(hp)
    ob[pl.ds(128, _L)] = lane_sums(ht)
    pltpu.sync_copy(ob, out_h.at[pl.ds(w * _PROW, _PROW)])


@functools.partial(
    pl.kernel,
    out_type=jax.ShapeDtypeStruct((_L,), jnp.float32),
    mesh=_mesh(),
    scratch_types=[
        pltpu.VMEM((_NW * _PROW,), jnp.float32),
        pltpu.VMEM((_L,), jnp.float32),
        pltpu.SemaphoreType.DMA,
    ],
    compiler_params=pltpu.CompilerParams(needs_layout_passes=False),
)
def _sc_finish(part_h, out_h, pv, ov, sem):
    w = lax.axis_index("s") * _NC + lax.axis_index("c")

    @pl.when(w == 0)
    def _():
        pltpu.sync_copy(part_h, pv)
        cnt = jnp.zeros((_L,), jnp.float32)
        ps = jnp.zeros((_L,), jnp.float32)
        ts = jnp.zeros((_L,), jnp.float32)
        for i in range(_NW):
            cnt = cnt + pv[pl.ds(i * _PROW, _L)]
            ps = ps + pv[pl.ds(i * _PROW + 64, _L)]
            ts = ts + pv[pl.ds(i * _PROW + 128, _L)]
        safe = jnp.maximum(cnt, 1.0)
        diff = jnp.abs(ps / safe - ts / safe)
        lv = jnp.where(cnt > 0.0, diff, 0.0) * jnp.float32(1.0 / _NBINS)
        total = jnp.sum(lv)
        ov[...] = jnp.zeros((_L,), jnp.float32) + total
        pltpu.sync_copy(ov, out_h)


def kernel(pred, target, input_img):
    partials = _sc_hist(input_img.reshape(-1), pred.reshape(-1),
                        target.reshape(-1))
    return _sc_finish(partials)[0]


# speedup vs baseline: 1.0386x; 1.0386x over previous
import functools

import jax
import jax.numpy as jnp
from jax import lax
from jax.experimental import pallas as pl
from jax.experimental.pallas import tpu as pltpu
from jax.experimental.pallas import tpu_sc as plsc

_PLANE = 512 * 512
_NBATCH = 16
_NPIX = _NBATCH * _PLANE
_NC, _NS, _L = 2, 16, 16
_NW = _NC * _NS
_PPW = _NPIX // _NW
_CH = 4096
_NCH = _PPW // _CH
_NV = _CH // _L
_NBINS = 16
_ROWS = _NBINS + 1
_PROW = 192


def _mesh():
    return plsc.VectorSubcoreMesh(
        core_axis_name="c", subcore_axis_name="s",
        num_cores=_NC, num_subcores=_NS)


@functools.partial(
    pl.kernel,
    out_type=jax.ShapeDtypeStruct((_NW * _PROW,), jnp.float32),
    mesh=_mesh(),
    scratch_types=[
        pltpu.VMEM((2 * 9 * _CH,), jnp.float32),
        pltpu.VMEM((_ROWS * _L,), jnp.float32),
        pltpu.VMEM((_ROWS * _L,), jnp.float32),
        pltpu.VMEM((_ROWS * _L,), jnp.float32),
        pltpu.VMEM((_PROW,), jnp.float32),
        pltpu.SemaphoreType.DMA,
        pltpu.SemaphoreType.DMA,
    ],
    compiler_params=pltpu.CompilerParams(needs_layout_passes=False),
)
def _sc_hist(inp_h, pred_h, targ_h, out_h, buf, hc, hp, ht, ob, sem0, sem1):
    w = lax.axis_index("s") * _NC + lax.axis_index("c")
    b = w // 2
    h = w % 2
    base = b * (3 * _PLANE) + h * _PPW
    sems = (sem0, sem1)
    arrs = (inp_h, pred_h, targ_h)

    zero = jnp.zeros((_L,), jnp.float32)
    for i in range(_ROWS):
        hc[pl.ds(i * _L, _L)] = zero
        hp[pl.ds(i * _L, _L)] = zero
        ht[pl.ds(i * _L, _L)] = zero
    for q in range(_PROW // _L):
        ob[pl.ds(q * _L, _L)] = zero

    def copies(g, ph):
        out = []
        for a in range(3):
            for c in range(3):
                src = arrs[a].at[pl.ds(base + c * _PLANE + g * _CH, _CH)]
                dst = buf.at[pl.ds((ph * 9 + a * 3 + c) * _CH, _CH)]
                out.append(pltpu.make_async_copy(src, dst, sems[ph]))
        return out

    def start(g, ph):
        for cp in copies(g, ph):
            cp.start()

    def drain(g, ph):
        for cp in copies(g, ph):
            cp.wait()

    lane = lax.iota(jnp.int32, _L)
    ones = jnp.ones((_L,), jnp.float32)

    def process(ph):
        def body(i, carry):
            def ld(j):
                return buf[pl.ds((ph * 9 + j) * _CH + i * _L, _L)]
            il = 0.299 * ld(0) + 0.587 * ld(1) + 0.114 * ld(2)
            addr = (il * 16.0).astype(jnp.int32) * _L + lane
            plu = 0.299 * ld(3) + 0.587 * ld(4) + 0.114 * ld(5)
            tlu = 0.299 * ld(6) + 0.587 * ld(7) + 0.114 * ld(8)
            plsc.addupdate_scatter(hc, [addr], ones)
            plsc.addupdate_scatter(hp, [addr], plu)
            plsc.addupdate_scatter(ht, [addr], tlu)
            return carry
        lax.fori_loop(0, _NV, body, 0, unroll=4)

    start(0, 0)
    start(1, 1)

    def outer(g0, carry):
        for ph in range(2):
            g = g0 * 2 + ph
            drain(g, ph)
            process(ph)

            @pl.when(g + 2 < _NCH)
            def _():
                start(g + 2, ph)
        return carry

    lax.fori_loop(0, _NCH // 2, outer, 0)

    def lane_sums(href):
        acc = jnp.zeros((_L,), jnp.float32)
        for n in range(_NBINS):
            acc = jnp.where(lane == n, jnp.sum(href[pl.ds(n * _L, _L)]), acc)
        return acc

    ob[pl.ds(0, _L)] = lane_sums(hc)
    ob[pl.ds(64, _L)] = lane_sums(hp)
    ob[pl.ds(128, _L)] = lane_sums(ht)
    pltpu.sync_copy(ob, out_h.at[pl.ds(w * _PROW, _PROW)])


@functools.partial(
    pl.kernel,
    out_type=jax.ShapeDtypeStruct((_L,), jnp.float32),
    mesh=_mesh(),
    scratch_types=[
        pltpu.VMEM((_NW * _PROW,), jnp.float32),
        pltpu.VMEM((_L,), jnp.float32),
        pltpu.SemaphoreType.DMA,
    ],
    compiler_params=pltpu.CompilerParams(needs_layout_passes=False),
)
def _sc_finish(part_h, out_h, pv, ov, sem):
    w = lax.axis_index("s") * _NC + lax.axis_index("c")

    @pl.when(w == 0)
    def _():
        pltpu.sync_copy(part_h, pv)
        cnt = jnp.zeros((_L,), jnp.float32)
        ps = jnp.zeros((_L,), jnp.float32)
        ts = jnp.zeros((_L,), jnp.float32)
        for i in range(_NW):
            cnt = cnt + pv[pl.ds(i * _PROW, _L)]
            ps = ps + pv[pl.ds(i * _PROW + 64, _L)]
            ts = ts + pv[pl.ds(i * _PROW + 128, _L)]
        safe = jnp.maximum(cnt, 1.0)
        diff = jnp.abs(ps / safe - ts / safe)
        lv = jnp.where(cnt > 0.0, diff, 0.0) * jnp.float32(1.0 / _NBINS)
        total = jnp.sum(lv)
        ov[...] = jnp.zeros((_L,), jnp.float32) + total
        pltpu.sync_copy(ov, out_h)


def kernel(pred, target, input_img):
    partials = _sc_hist(input_img.reshape(-1), pred.reshape(-1),
                        target.reshape(-1))
    return _sc_finish(partials)[0]


# speedup vs baseline: 1.6747x; 1.6123x over previous
import functools

import jax
import jax.numpy as jnp
from jax import lax
from jax.experimental import pallas as pl
from jax.experimental.pallas import tpu as pltpu
from jax.experimental.pallas import tpu_sc as plsc

_H = 512
_W = 512
_NBATCH = 16
_NC, _NS, _L = 2, 16, 16
_NW = _NC * _NS
_RPW = _H // 2
_CR = 8
_NCH = _RPW // _CR
_NV = _CR * _W // _L
_VPR = _W // _L
_NBINS = 16
_ROWS = _NBINS + 1
_PROW = 192


def _mesh():
    return plsc.VectorSubcoreMesh(
        core_axis_name="c", subcore_axis_name="s",
        num_cores=_NC, num_subcores=_NS)


_SCRATCH = (
    [pltpu.VMEM((_CR, _W), jnp.float32) for _ in range(18)]
    + [
        pltpu.VMEM((_ROWS * _L,), jnp.float32),
        pltpu.VMEM((_ROWS * _L,), jnp.float32),
        pltpu.VMEM((_ROWS * _L,), jnp.float32),
        pltpu.VMEM((_PROW,), jnp.float32),
        pltpu.SemaphoreType.DMA,
        pltpu.SemaphoreType.DMA,
    ]
)


@functools.partial(
    pl.kernel,
    out_type=jax.ShapeDtypeStruct((_NW * _PROW,), jnp.float32),
    mesh=_mesh(),
    scratch_types=_SCRATCH,
    compiler_params=pltpu.CompilerParams(needs_layout_passes=False),
)
def _sc_hist(inp_h, pred_h, targ_h, out_h, *rest):
    bufs = rest[:18]
    hc, hp, ht, ob, sem0, sem1 = rest[18:]
    w = lax.axis_index("s") * _NC + lax.axis_index("c")
    b = w // 2
    h = w % 2
    row_base = h * _RPW
    sems = (sem0, sem1)
    arrs = (inp_h, pred_h, targ_h)

    zero = jnp.zeros((_L,), jnp.float32)
    for i in range(_ROWS):
        hc[pl.ds(i * _L, _L)] = zero
        hp[pl.ds(i * _L, _L)] = zero
        ht[pl.ds(i * _L, _L)] = zero
    for q in range(_PROW // _L):
        ob[pl.ds(q * _L, _L)] = zero

    def copies(g, ph):
        out = []
        for a in range(3):
            for c in range(3):
                src = arrs[a].at[b, c, pl.ds(row_base + g * _CR, _CR), :]
                out.append(pltpu.make_async_copy(
                    src, bufs[ph * 9 + a * 3 + c], sems[ph]))
        return out

    def start(g, ph):
        for cp in copies(g, ph):
            cp.start()

    def drain(g, ph):
        for cp in copies(g, ph):
            cp.wait()

    lane = lax.iota(jnp.int32, _L)
    ones = jnp.ones((_L,), jnp.float32)

    def process(ph):
        def body(i, carry):
            r = i // _VPR
            c0 = (i % _VPR) * _L
            ir = jnp.zeros((_L,), jnp.int32) + r
            ic = lane + c0

            def ld(j):
                return plsc.load_gather(bufs[ph * 9 + j], [ir, ic])
            il = 0.299 * ld(0) + 0.587 * ld(1) + 0.114 * ld(2)
            addr = (il * 16.0).astype(jnp.int32) * _L + lane
            plu = 0.299 * ld(3) + 0.587 * ld(4) + 0.114 * ld(5)
            tlu = 0.299 * ld(6) + 0.587 * ld(7) + 0.114 * ld(8)
            plsc.addupdate_scatter(hc, [addr], ones)
            plsc.addupdate_scatter(hp, [addr], plu)
            plsc.addupdate_scatter(ht, [addr], tlu)
            return carry
        lax.fori_loop(0, _NV, body, 0, unroll=4)

    start(0, 0)
    start(1, 1)

    def outer(g0, carry):
        for ph in range(2):
            g = g0 * 2 + ph
            drain(g, ph)
            process(ph)

            @pl.when(g + 2 < _NCH)
            def _():
                start(g + 2, ph)
        return carry

    lax.fori_loop(0, _NCH // 2, outer, 0)

    def lane_sums(href):
        acc = jnp.zeros((_L,), jnp.float32)
        for n in range(_NBINS):
            acc = jnp.where(lane == n, jnp.sum(href[pl.ds(n * _L, _L)]), acc)
        return acc

    ob[pl.ds(0, _L)] = lane_sums(hc)
    ob[pl.ds(64, _L)] = lane_sums(hp)
    ob[pl.ds(128, _L)] = lane_sums(ht)
    pltpu.sync_copy(ob, out_h.at[pl.ds(w * _PROW, _PROW)])


@functools.partial(
    pl.kernel,
    out_type=jax.ShapeDtypeStruct((_L,), jnp.float32),
    mesh=_mesh(),
    scratch_types=[
        pltpu.VMEM((_NW * _PROW,), jnp.float32),
        pltpu.VMEM((_L,), jnp.float32),
        pltpu.SemaphoreType.DMA,
    ],
    compiler_params=pltpu.CompilerParams(needs_layout_passes=False),
)
def _sc_finish(part_h, out_h, pv, ov, sem):
    w = lax.axis_index("s") * _NC + lax.axis_index("c")

    @pl.when(w == 0)
    def _():
        pltpu.sync_copy(part_h, pv)
        cnt = jnp.zeros((_L,), jnp.float32)
        ps = jnp.zeros((_L,), jnp.float32)
        ts = jnp.zeros((_L,), jnp.float32)
        for i in range(_NW):
            cnt = cnt + pv[pl.ds(i * _PROW, _L)]
            ps = ps + pv[pl.ds(i * _PROW + 64, _L)]
            ts = ts + pv[pl.ds(i * _PROW + 128, _L)]
        safe = jnp.maximum(cnt, 1.0)
        diff = jnp.abs(ps / safe - ts / safe)
        lv = jnp.where(cnt > 0.0, diff, 0.0) * jnp.float32(1.0 / _NBINS)
        total = jnp.sum(lv)
        ov[...] = jnp.zeros((_L,), jnp.float32) + total
        pltpu.sync_copy(ov, out_h)


def kernel(pred, target, input_img):
    partials = _sc_hist(input_img, pred, target)
    return _sc_finish(partials)[0]


# speedup vs baseline: 3.4820x; 2.0793x over previous
import functools

import jax
import jax.numpy as jnp
from jax import lax
from jax.experimental import pallas as pl
from jax.experimental.pallas import tpu as pltpu
from jax.experimental.pallas import tpu_sc as plsc

_H = 512
_W = 512
_NBATCH = 16
_SCB = 12
_NC, _NS, _L = 2, 16, 16
_NW = _NC * _NS
_RPW = _SCB * _H // _NW
_CR = 8
_NCH = _RPW // _CR
_NV = _CR * _W // _L
_VPR = _W // _L
_NBINS = 16
_ROWS = _NBINS + 1
_PROW = 192
_NBUF = 3


def _mesh():
    return plsc.VectorSubcoreMesh(
        core_axis_name="c", subcore_axis_name="s",
        num_cores=_NC, num_subcores=_NS)


_SCRATCH = (
    [pltpu.VMEM((_CR, _W), jnp.float32) for _ in range(9 * _NBUF)]
    + [
        pltpu.VMEM((_ROWS * _L,), jnp.float32),
        pltpu.VMEM((_ROWS * _L,), jnp.float32),
        pltpu.VMEM((_ROWS * _L,), jnp.float32),
        pltpu.VMEM((_PROW,), jnp.float32),
    ]
    + [pltpu.SemaphoreType.DMA for _ in range(_NBUF)]
)


@functools.partial(
    pl.kernel,
    out_type=jax.ShapeDtypeStruct((_NW * _PROW,), jnp.float32),
    mesh=_mesh(),
    scratch_types=_SCRATCH,
    compiler_params=pltpu.CompilerParams(needs_layout_passes=False),
)
def _sc_hist(inp_h, pred_h, targ_h, out_h, *rest):
    bufs = rest[:9 * _NBUF]
    hc, hp, ht, ob = rest[9 * _NBUF:9 * _NBUF + 4]
    sems = rest[9 * _NBUF + 4:]
    w = lax.axis_index("s") * _NC + lax.axis_index("c")
    glob0 = w * _RPW
    arrs = (inp_h, pred_h, targ_h)

    zero = jnp.zeros((_L,), jnp.float32)
    for i in range(_ROWS):
        hc[pl.ds(i * _L, _L)] = zero
        hp[pl.ds(i * _L, _L)] = zero
        ht[pl.ds(i * _L, _L)] = zero
    for q in range(_PROW // _L):
        ob[pl.ds(q * _L, _L)] = zero

    def copies(g, ph):
        glob = glob0 + g * _CR
        bb = glob // _H
        r0 = glob % _H
        out = []
        for a in range(3):
            for c in range(3):
                src = arrs[a].at[bb, c, pl.ds(r0, _CR), :]
                out.append(pltpu.make_async_copy(
                    src, bufs[ph * 9 + a * 3 + c], sems[ph]))
        return out

    def start(g, ph):
        for cp in copies(g, ph):
            cp.start()

    def drain(g, ph):
        for cp in copies(g, ph):
            cp.wait()

    lane = lax.iota(jnp.int32, _L)
    ones = jnp.ones((_L,), jnp.float32)

    def process(ph):
        @plsc.parallel_loop(0, _NV, unroll=8)
        def body(i):
            r = i // _VPR
            c0 = (i % _VPR) * _L
            ir = jnp.zeros((_L,), jnp.int32) + r
            ic = lane + c0

            def ld(j):
                return plsc.load_gather(bufs[ph * 9 + j], [ir, ic])
            il = 0.299 * ld(0) + 0.587 * ld(1) + 0.114 * ld(2)
            addr = (il * 16.0).astype(jnp.int32) * _L + lane
            plu = 0.299 * ld(3) + 0.587 * ld(4) + 0.114 * ld(5)
            tlu = 0.299 * ld(6) + 0.587 * ld(7) + 0.114 * ld(8)
            plsc.addupdate_scatter(hc, [addr], ones)
            plsc.addupdate_scatter(hp, [addr], plu)
            plsc.addupdate_scatter(ht, [addr], tlu)

    for ph in range(_NBUF):
        start(ph, ph)

    def outer(g0, carry):
        for ph in range(_NBUF):
            g = g0 * _NBUF + ph
            drain(g, ph)
            process(ph)

            @pl.when(g + _NBUF < _NCH)
            def _():
                start(g + _NBUF, ph)
        return carry

    lax.fori_loop(0, _NCH // _NBUF, outer, 0)

    def lane_sums(href):
        acc = jnp.zeros((_L,), jnp.float32)
        for n in range(_NBINS):
            acc = jnp.where(lane == n, jnp.sum(href[pl.ds(n * _L, _L)]), acc)
        return acc

    ob[pl.ds(0, _L)] = lane_sums(hc)
    ob[pl.ds(64, _L)] = lane_sums(hp)
    ob[pl.ds(128, _L)] = lane_sums(ht)
    pltpu.sync_copy(ob, out_h.at[pl.ds(w * _PROW, _PROW)])


def _tc_body(inp_ref, pred_ref, targ_ref, o_ref):
    step = pl.program_id(0)

    @pl.when(step == 0)
    def _():
        for k in range(48):
            o_ref[k] = 0.0

    x = inp_ref[0]
    il = 0.299 * x[0] + 0.587 * x[1] + 0.114 * x[2]
    p = pred_ref[0]
    plu = 0.299 * p[0] + 0.587 * p[1] + 0.114 * p[2]
    t = targ_ref[0]
    tlu = 0.299 * t[0] + 0.587 * t[1] + 0.114 * t[2]
    bi = (il * 16.0).astype(jnp.int32)
    for i in range(_NBINS):
        m = bi == i
        o_ref[i] = o_ref[i] + jnp.sum(jnp.where(m, 1.0, 0.0))
        o_ref[16 + i] = o_ref[16 + i] + jnp.sum(jnp.where(m, plu, 0.0))
        o_ref[32 + i] = o_ref[32 + i] + jnp.sum(jnp.where(m, tlu, 0.0))


_TCROWBLOCKS = _H // 128
_TC_GRID = (_NBATCH - _SCB) * _TCROWBLOCKS

_tc_hist = pl.pallas_call(
    _tc_body,
    grid=(_TC_GRID,),
    in_specs=[
        pl.BlockSpec((1, 3, 128, _W),
                     lambda i: (_SCB + i // _TCROWBLOCKS, 0,
                                i % _TCROWBLOCKS, 0))
        for _ in range(3)
    ],
    out_specs=pl.BlockSpec(memory_space=pltpu.SMEM),
    out_shape=jax.ShapeDtypeStruct((48,), jnp.float32),
    compiler_params=pltpu.CompilerParams(
        dimension_semantics=("arbitrary",)),
)


@functools.partial(
    pl.kernel,
    out_type=jax.ShapeDtypeStruct((_L,), jnp.float32),
    mesh=_mesh(),
    scratch_types=[
        pltpu.VMEM((_NW * _PROW,), jnp.float32),
        pltpu.VMEM((48,), jnp.float32),
        pltpu.VMEM((_L,), jnp.float32),
    ],
    compiler_params=pltpu.CompilerParams(needs_layout_passes=False),
)
def _sc_finish(part_h, tc_h, out_h, pv, tv, ov):
    w = lax.axis_index("s") * _NC + lax.axis_index("c")

    @pl.when(w == 0)
    def _():
        pltpu.sync_copy(part_h, pv)
        pltpu.sync_copy(tc_h, tv)
        cnt = tv[pl.ds(0, _L)]
        ps = tv[pl.ds(16, _L)]
        ts = tv[pl.ds(32, _L)]
        for i in range(_NW):
            cnt = cnt + pv[pl.ds(i * _PROW, _L)]
            ps = ps + pv[pl.ds(i * _PROW + 64, _L)]
            ts = ts + pv[pl.ds(i * _PROW + 128, _L)]
        safe = jnp.maximum(cnt, 1.0)
        diff = jnp.abs(ps / safe - ts / safe)
        lv = jnp.where(cnt > 0.0, diff, 0.0) * jnp.float32(1.0 / _NBINS)
        ov[...] = jnp.zeros((_L,), jnp.float32) + jnp.sum(lv)
        pltpu.sync_copy(ov, out_h)


def kernel(pred, target, input_img):
    sc_part = _sc_hist(input_img, pred, target)
    tc_part = _tc_hist(input_img, pred, target)
    return _sc_finish(sc_part, tc_part)[0]
